# Initial kernel scaffold; baseline (speedup 1.0000x reference)
import jax
import jax.numpy as jnp
from jax.experimental import pallas as pl


def kernel(x):
    raise NotImplementedError("write your pallas kernel here")


# speedup vs baseline: 186.0961x; 186.0961x over previous
import functools

import jax
import jax.numpy as jnp
from jax import lax
from jax.experimental import pallas as pl
from jax.experimental.pallas import tpu as pltpu

F = 1025
T = 256
B = 2
MAXP = 5
MARGIN = 3
PWR = 0.1
LLIM = F - (MARGIN + 1)


def _tc_body(x_ref, o_ref):
    a = x_ref[:, 1:, :]
    rows = lax.broadcasted_iota(jnp.int32, a.shape, 1)
    work = a
    theta = None
    for _ in range(MAXP):
        mj = jnp.max(work, axis=1, keepdims=True)
        hit = work == mj
        r = jnp.min(jnp.where(hit, rows, F), axis=1, keepdims=True)
        work = jnp.where(rows == r, -jnp.inf, work)
        theta = mj
    ok = (a >= theta) & (a > PWR)
    f0 = jnp.min(jnp.where(ok, rows + 1, F), axis=1, keepdims=True)
    f0 = jnp.where(f0 == F, 0, f0)
    f0f = f0.astype(jnp.float32)
    safe = jnp.maximum(f0f, 1.0)
    kk = lax.broadcasted_iota(jnp.int32, (B, F, T), 1).astype(jnp.float32)
    mmax = jnp.floor(jnp.float32(LLIM - 1) / safe)
    m = jnp.minimum(mmax, jnp.floor((kk + MARGIN) / safe))
    d = jnp.abs(kk - m * f0f)
    cover = (f0f > 0.0) & (m >= 1.0) & (d <= MARGIN)
    val = jnp.maximum(1.0 - d * (0.5 / MARGIN), 0.5)
    o_ref[...] = jnp.where(cover, val, jnp.float32(0.5))


@functools.partial(jax.jit, static_argnames=("interpret",))
def _tc_mask(x2, interpret=False):
    return pl.pallas_call(
        _tc_body,
        out_shape=jax.ShapeDtypeStruct((B, F, T), jnp.float32),
        interpret=interpret,
    )(x2)


def kernel(x):
    x2 = x.reshape(B, F, T)
    return _tc_mask(x2).reshape(B, 1, F, T)
